# baseline (speedup 1.0000x reference)
import jax
import jax.numpy as jnp
from jax import lax
from jax.experimental import pallas as pl
from jax.experimental.pallas import tpu as pltpu
from jax.experimental.pallas import tpu_sc as plsc

_C = 21
_RW = 256
_LN = 512

_NC = 2
_NW = 32
_SCB = 2
_TCB = 8 - _SCB
_SPT = 2
_HB = 448


def _tc_body(pred_ref, gt_ref, out_ref):
    b = pl.program_id(0)
    j = pl.program_id(1)

    x = pred_ref[0]
    best = x[0]
    bidx = jnp.zeros((_RW, _LN), jnp.int32)
    for c in range(1, _C):
        xc = x[c]
        upd = xc > best
        best = jnp.where(upd, xc, best)
        bidx = jnp.where(upd, c, bidx)

    gt = gt_ref[0]

    iota2 = lax.broadcasted_iota(jnp.int32, (_C, _LN), 0)
    acc = jnp.zeros((_C, _C), jnp.float32)
    for r0 in range(0, _RW, 8):
        ohg = jnp.concatenate(
            [(iota2 == gt[r : r + 1]).astype(jnp.float32) for r in range(r0, r0 + 8)],
            axis=1,
        )
        ohp = jnp.concatenate(
            [(iota2 == bidx[r : r + 1]).astype(jnp.float32) for r in range(r0, r0 + 8)],
            axis=1,
        )
        acc = acc + lax.dot_general(
            ohg,
            ohp,
            (((1,), (1,)), ((), ())),
            preferred_element_type=jnp.float32,
        )

    @pl.when(jnp.logical_and(b == 0, j == 0))
    def _():
        out_ref[...] = jnp.zeros_like(out_ref)

    out_ref[...] += acc


def _sc_body(pred_hbm, gt_hbm, out_hbm, pred_v, gt_v, hist, sem):
    wid = lax.axis_index("s") * _NC + lax.axis_index("c")

    def _zero(i, carry):
        hist[pl.ds(i * 16, 16)] = jnp.zeros((16,), jnp.int32)
        return carry

    lax.fori_loop(0, _HB, _zero, 0)

    lane = lax.iota(jnp.int32, 16)
    ones = jnp.ones((16,), jnp.int32)

    for bb in range(_SCB):
        b = _TCB + bb
        for s2 in range(_SPT):
            st = wid * _SPT + s2
            r0 = st * 8
            cp = pltpu.async_copy(
                pred_hbm.at[b, :, pl.ds(r0, 8), :], pred_v, sem
            )
            cg = pltpu.async_copy(gt_hbm.at[b, pl.ds(r0, 8), :], gt_v, sem)
            cp.wait()
            cg.wait()

            def _row(r, carry):
                def _grp(j, carry2):
                    off = j * 16
                    best = pred_v[0, r, pl.ds(off, 16)]
                    bidx = jnp.zeros((16,), jnp.int32)
                    for c in range(1, _C):
                        v = pred_v[c, r, pl.ds(off, 16)]
                        m = v > best
                        best = jnp.where(m, v, best)
                        bidx = jnp.where(m, jnp.full((16,), c, jnp.int32), bidx)
                    g = gt_v[r, pl.ds(off, 16)]
                    idx = (g * _C + bidx) * 16 + lane
                    plsc.addupdate_scatter(hist, [idx], ones)
                    return carry2

                lax.fori_loop(0, _LN // 16, _grp, carry)
                return carry

            lax.fori_loop(0, 8, _row, 0)

    pltpu.sync_copy(hist, out_hbm.at[wid])


@jax.jit
def kernel(prediction, groundtruth):
    tc_out = pl.pallas_call(
        _tc_body,
        grid=(_TCB, 512 // _RW),
        in_specs=[
            pl.BlockSpec((1, _C, _RW, _LN), lambda b, j: (b, 0, j, 0)),
            pl.BlockSpec((1, _RW, _LN), lambda b, j: (b, j, 0)),
        ],
        out_specs=pl.BlockSpec((_C, _C), lambda b, j: (0, 0)),
        out_shape=jax.ShapeDtypeStruct((_C, _C), jnp.float32),
    )(prediction, groundtruth)

    sc_out = pl.kernel(
        _sc_body,
        out_type=jax.ShapeDtypeStruct((_NW, _HB * 16), jnp.int32),
        mesh=plsc.VectorSubcoreMesh(core_axis_name="c", subcore_axis_name="s"),
        compiler_params=pltpu.CompilerParams(needs_layout_passes=False),
        scratch_types=[
            pltpu.VMEM((_C, 8, _LN), jnp.float32),
            pltpu.VMEM((8, _LN), jnp.int32),
            pltpu.VMEM((_HB * 16,), jnp.int32),
            pltpu.SemaphoreType.DMA,
        ],
    )(prediction, groundtruth)

    sc_part = (
        sc_out.sum(axis=0).reshape(_HB, 16).sum(axis=1)[: _C * _C].reshape(_C, _C)
    )
    return (tc_out + sc_part.astype(jnp.float32)).astype(jnp.int32)


# speedup vs baseline: 1.1064x; 1.1064x over previous
import jax
import jax.numpy as jnp
from jax import lax
from jax.experimental import pallas as pl
from jax.experimental.pallas import tpu as pltpu
from jax.experimental.pallas import tpu_sc as plsc

_C = 21
_RW = 256
_LN = 512

_NC = 2
_NW = 32
_SCB = 2
_TCB = 8 - _SCB
_SPT = 2
_HB = 448


def _tc_body(pred_ref, gt_ref, out_ref):
    b = pl.program_id(0)
    j = pl.program_id(1)

    x = pred_ref[0]
    best = x[0]
    bidx = jnp.zeros((_RW, _LN), jnp.int32)
    for c in range(1, _C):
        xc = x[c]
        upd = xc > best
        best = jnp.where(upd, xc, best)
        bidx = jnp.where(upd, c, bidx)

    gt = gt_ref[0]

    iota2 = lax.broadcasted_iota(jnp.int32, (_C, _LN), 0)
    acc = jnp.zeros((_C, _C), jnp.float32)
    for r0 in range(0, _RW, 8):
        ohg = jnp.concatenate(
            [(iota2 == gt[r : r + 1]).astype(jnp.float32) for r in range(r0, r0 + 8)],
            axis=1,
        )
        ohp = jnp.concatenate(
            [(iota2 == bidx[r : r + 1]).astype(jnp.float32) for r in range(r0, r0 + 8)],
            axis=1,
        )
        acc = acc + lax.dot_general(
            ohg,
            ohp,
            (((1,), (1,)), ((), ())),
            preferred_element_type=jnp.float32,
        )

    @pl.when(jnp.logical_and(b == 0, j == 0))
    def _():
        out_ref[...] = jnp.zeros_like(out_ref)

    out_ref[...] += acc


_HR = 4


def _sc_body(pred_hbm, gt_hbm, out_hbm, pv0, gv0, pv1, gv1, hist, sem0, sem1):
    wid = lax.axis_index("s") * _NC + lax.axis_index("c")

    def _zero(i, carry):
        hist[pl.ds(i * 16, 16)] = jnp.zeros((16,), jnp.int32)
        return carry

    lax.fori_loop(0, _HB, _zero, 0)

    lane = lax.iota(jnp.int32, 16)
    ones = jnp.ones((16,), jnp.int32)
    bufs = ((pv0, gv0, sem0), (pv1, gv1, sem1))

    chunks = []
    for bb in range(_SCB):
        for s2 in range(_SPT):
            for h in range(2):
                chunks.append((_TCB + bb, (wid * _SPT + s2) * 8 + h * _HR))

    def _start(chunk, buf):
        b, r0 = chunk
        pv, gv, sem = buf
        cp = pltpu.async_copy(pred_hbm.at[b, :, pl.ds(r0, _HR), :], pv, sem)
        cg = pltpu.async_copy(gt_hbm.at[b, pl.ds(r0, _HR), :], gv, sem)
        return cp, cg

    def _compute(buf):
        pv, gv, _ = buf

        def _row(r, carry):
            def _grp(j, carry2):
                off = j * 16
                best = pv[0, r, pl.ds(off, 16)]
                bidx = jnp.zeros((16,), jnp.int32)
                for c in range(1, _C):
                    v = pv[c, r, pl.ds(off, 16)]
                    m = v > best
                    best = jnp.where(m, v, best)
                    bidx = jnp.where(m, jnp.full((16,), c, jnp.int32), bidx)
                g = gv[r, pl.ds(off, 16)]
                idx = (g * _C + bidx) * 16 + lane
                plsc.addupdate_scatter(hist, [idx], ones)
                return carry2

            lax.fori_loop(0, _LN // 16, _grp, carry)
            return carry

        lax.fori_loop(0, _HR, _row, 0)

    pending = _start(chunks[0], bufs[0])
    for i in range(len(chunks)):
        if i + 1 < len(chunks):
            nxt = _start(chunks[i + 1], bufs[(i + 1) % 2])
        pending[0].wait()
        pending[1].wait()
        _compute(bufs[i % 2])
        if i + 1 < len(chunks):
            pending = nxt

    pltpu.sync_copy(hist, out_hbm.at[wid])


@jax.jit
def kernel(prediction, groundtruth):
    tc_out = pl.pallas_call(
        _tc_body,
        grid=(_TCB, 512 // _RW),
        in_specs=[
            pl.BlockSpec((1, _C, _RW, _LN), lambda b, j: (b, 0, j, 0)),
            pl.BlockSpec((1, _RW, _LN), lambda b, j: (b, j, 0)),
        ],
        out_specs=pl.BlockSpec((_C, _C), lambda b, j: (0, 0)),
        out_shape=jax.ShapeDtypeStruct((_C, _C), jnp.float32),
    )(prediction, groundtruth)

    sc_out = pl.kernel(
        _sc_body,
        out_type=jax.ShapeDtypeStruct((_NW, _HB * 16), jnp.int32),
        mesh=plsc.VectorSubcoreMesh(core_axis_name="c", subcore_axis_name="s"),
        compiler_params=pltpu.CompilerParams(needs_layout_passes=False),
        scratch_types=[
            pltpu.VMEM((_C, _HR, _LN), jnp.float32),
            pltpu.VMEM((_HR, _LN), jnp.int32),
            pltpu.VMEM((_C, _HR, _LN), jnp.float32),
            pltpu.VMEM((_HR, _LN), jnp.int32),
            pltpu.VMEM((_HB * 16,), jnp.int32),
            pltpu.SemaphoreType.DMA,
            pltpu.SemaphoreType.DMA,
        ],
    )(prediction, groundtruth)

    sc_part = (
        sc_out.sum(axis=0).reshape(_HB, 16).sum(axis=1)[: _C * _C].reshape(_C, _C)
    )
    return (tc_out + sc_part.astype(jnp.float32)).astype(jnp.int32)
